# baseline (speedup 1.0000x reference)
import functools

import jax
import jax.numpy as jnp
from jax.experimental import pallas as pl
from jax.experimental.pallas import tpu as pltpu


def _body(x_ref, plo_ref, phi_ref, elo_ref, ehi_ref, ptab_ref, etab_ref,
          out_ref):
    nb = x_ref.shape[0]
    for i in range(nb):
        vp = x_ref[i, 0:1, :]
        ve = x_ref[i, 1:2, :]
        oh_p = ((plo_ref[:, :] < vp)
                & (phi_ref[:, :] >= vp)).astype(jnp.bfloat16)
        oh_e = ((elo_ref[:, :] < ve)
                & (ehi_ref[:, :] >= ve)).astype(jnp.bfloat16)
        out_ref[i] = (
            jnp.dot(ptab_ref[:, :], oh_p, preferred_element_type=jnp.float32)
            + jnp.dot(etab_ref[:, :], oh_e, preferred_element_type=jnp.float32)
        )


@functools.partial(jax.jit, static_argnames=("interpret",))
def kernel(x, pitch_bins, energy_bins, pitch_embedding, energy_embedding,
           interpret=False):
    B, _, T = x.shape
    N, H = pitch_embedding.shape
    Tt = 2048
    Bb = 8

    inf = jnp.array([jnp.inf], dtype=jnp.float32)
    p_lo = jnp.concatenate([-inf, pitch_bins])[:, None]
    p_hi = jnp.concatenate([pitch_bins, inf])[:, None]
    e_lo = jnp.concatenate([-inf, energy_bins])[:, None]
    e_hi = jnp.concatenate([energy_bins, inf])[:, None]
    ptab = pitch_embedding.T.astype(jnp.bfloat16)
    etab = energy_embedding.T.astype(jnp.bfloat16)

    grid = (B // Bb, T // Tt)
    return pl.pallas_call(
        _body,
        grid=grid,
        in_specs=[
            pl.BlockSpec((Bb, 2, Tt), lambda b, j: (b, 0, j)),
            pl.BlockSpec((N, 1), lambda b, j: (0, 0)),
            pl.BlockSpec((N, 1), lambda b, j: (0, 0)),
            pl.BlockSpec((N, 1), lambda b, j: (0, 0)),
            pl.BlockSpec((N, 1), lambda b, j: (0, 0)),
            pl.BlockSpec((H, N), lambda b, j: (0, 0)),
            pl.BlockSpec((H, N), lambda b, j: (0, 0)),
        ],
        out_specs=pl.BlockSpec((Bb, H, Tt), lambda b, j: (b, 0, j)),
        out_shape=jax.ShapeDtypeStruct((B, H, T), jnp.float32),
        compiler_params=pltpu.CompilerParams(
            dimension_semantics=("parallel", "parallel")),
        interpret=interpret,
    )(x, p_lo, p_hi, e_lo, e_hi, ptab, etab)


# speedup vs baseline: 1.0239x; 1.0239x over previous
import functools

import jax
import jax.numpy as jnp
from jax.experimental import pallas as pl
from jax.experimental.pallas import tpu as pltpu


def _body(x_ref, plo_ref, phi_ref, elo_ref, ehi_ref, ctab_ref, out_ref):
    nb = x_ref.shape[0]
    for i in range(nb):
        vp = x_ref[i, 0:1, :]
        ve = x_ref[i, 1:2, :]
        oh_p = ((plo_ref[:, :] < vp)
                & (phi_ref[:, :] >= vp)).astype(jnp.bfloat16)
        oh_e = ((elo_ref[:, :] < ve)
                & (ehi_ref[:, :] >= ve)).astype(jnp.bfloat16)
        oh = jnp.concatenate([oh_p, oh_e], axis=0)
        out_ref[i] = jnp.dot(ctab_ref[:, :], oh,
                             preferred_element_type=jnp.float32)


@functools.partial(jax.jit, static_argnames=("interpret",))
def kernel(x, pitch_bins, energy_bins, pitch_embedding, energy_embedding,
           interpret=False):
    B, _, T = x.shape
    N, H = pitch_embedding.shape
    Tt = 2048
    Bb = 4

    inf = jnp.array([jnp.inf], dtype=jnp.float32)
    p_lo = jnp.concatenate([-inf, pitch_bins])[:, None]
    p_hi = jnp.concatenate([pitch_bins, inf])[:, None]
    e_lo = jnp.concatenate([-inf, energy_bins])[:, None]
    e_hi = jnp.concatenate([energy_bins, inf])[:, None]
    ctab = jnp.concatenate(
        [pitch_embedding.T, energy_embedding.T], axis=1,
    ).astype(jnp.bfloat16)

    grid = (B // Bb, T // Tt)
    return pl.pallas_call(
        _body,
        grid=grid,
        in_specs=[
            pl.BlockSpec((Bb, 2, Tt), lambda b, j: (b, 0, j)),
            pl.BlockSpec((N, 1), lambda b, j: (0, 0)),
            pl.BlockSpec((N, 1), lambda b, j: (0, 0)),
            pl.BlockSpec((N, 1), lambda b, j: (0, 0)),
            pl.BlockSpec((N, 1), lambda b, j: (0, 0)),
            pl.BlockSpec((H, 2 * N), lambda b, j: (0, 0)),
        ],
        out_specs=pl.BlockSpec((Bb, H, Tt), lambda b, j: (b, 0, j)),
        out_shape=jax.ShapeDtypeStruct((B, H, T), jnp.float32),
        compiler_params=pltpu.CompilerParams(
            dimension_semantics=("parallel", "parallel")),
        interpret=interpret,
    )(x, p_lo, p_hi, e_lo, e_hi, ctab)


# speedup vs baseline: 1.1248x; 1.0985x over previous
import functools

import jax
import jax.numpy as jnp
from jax.experimental import pallas as pl
from jax.experimental.pallas import tpu as pltpu


def _body(x_ref, phi_ref, ehi_ref, ctab_ref, out_ref):
    nb = x_ref.shape[0]
    zrow = jnp.zeros((1, x_ref.shape[2]), dtype=jnp.bfloat16)
    for i in range(nb):
        vp = x_ref[i, 0:1, :]
        ve = x_ref[i, 1:2, :]
        g_p = (phi_ref[:, :] >= vp).astype(jnp.bfloat16)
        g_e = (ehi_ref[:, :] >= ve).astype(jnp.bfloat16)
        oh_p = g_p - jnp.concatenate([zrow, g_p[:-1, :]], axis=0)
        oh_e = g_e - jnp.concatenate([zrow, g_e[:-1, :]], axis=0)
        oh = jnp.concatenate([oh_p, oh_e], axis=0)
        out_ref[i] = jnp.dot(ctab_ref[:, :], oh,
                             preferred_element_type=jnp.float32)


@functools.partial(jax.jit, static_argnames=("interpret",))
def kernel(x, pitch_bins, energy_bins, pitch_embedding, energy_embedding,
           interpret=False):
    B, _, T = x.shape
    N, H = pitch_embedding.shape
    Tt = 2048
    Bb = 4

    inf = jnp.array([jnp.inf], dtype=jnp.float32)
    p_hi = jnp.concatenate([pitch_bins, inf])[:, None]
    e_hi = jnp.concatenate([energy_bins, inf])[:, None]
    ctab = jnp.concatenate(
        [pitch_embedding.T, energy_embedding.T], axis=1,
    ).astype(jnp.bfloat16)

    grid = (B // Bb, T // Tt)
    return pl.pallas_call(
        _body,
        grid=grid,
        in_specs=[
            pl.BlockSpec((Bb, 2, Tt), lambda b, j: (b, 0, j)),
            pl.BlockSpec((N, 1), lambda b, j: (0, 0)),
            pl.BlockSpec((N, 1), lambda b, j: (0, 0)),
            pl.BlockSpec((H, 2 * N), lambda b, j: (0, 0)),
        ],
        out_specs=pl.BlockSpec((Bb, H, Tt), lambda b, j: (b, 0, j)),
        out_shape=jax.ShapeDtypeStruct((B, H, T), jnp.float32),
        compiler_params=pltpu.CompilerParams(
            dimension_semantics=("parallel", "parallel")),
        interpret=interpret,
    )(x, p_hi, e_hi, ctab)
